# baseline (speedup 1.0000x reference)
import functools

import jax
import jax.numpy as jnp
import numpy as np
from jax import lax
from jax.experimental import pallas as pl
from jax.experimental.pallas import tpu as pltpu
from jax.experimental.pallas import tpu_sc as plsc

N = 10000
E = 320000
F = 128
G = 64
NC = 2
NS = 16
K = 128
CHUNKS = (-(-E // (NC * NS * K)) + 3) // 4 * 4
CH0 = 120
CH1 = 2 * CHUNKS - CH0
EPAD = NC * NS * K * CHUNKS
NPAD = N + 112
ROWS_PER_TILE = NPAD // NS

@functools.cache
def _sc_mesh():
    return plsc.VectorSubcoreMesh(core_axis_name="c", subcore_axis_name="s")


def _zero_fill(ref, nrows, ncols):
    zeros16 = jnp.zeros((16,), jnp.float32)

    def body(i, _):
        ref[i // (ncols // 16), pl.ds((i % (ncols // 16)) * 16, 16)] = zeros16
        return 0

    lax.fori_loop(0, nrows * (ncols // 16), body, 0)


def _zero_fill3(ref, n0, n1):
    zeros16 = jnp.zeros((16,), jnp.float32)

    def body(i, _):
        ref[i // n1, i % n1, :] = zeros16
        return 0

    lax.fori_loop(0, n0 * n1, body, 0)


def _zero_acc_slice(acc, rows_buf, tile, width):
    base = tile * ROWS_PER_TILE
    done = 0
    while done < ROWS_PER_TILE:
        nr = min(K, ROWS_PER_TILE - done)
        pltpu.sync_copy(rows_buf.at[pl.ds(0, nr)], acc.at[pl.ds(base + done, nr)])
        done += nr


def _deg_body(dstp_hbm, out_hbm, acc, didx, vals, sem):
    c = lax.axis_index("c")
    s = lax.axis_index("s")
    wid = c * NS + s
    _zero_fill(vals, K, 16)
    _zero_acc_slice(acc, vals, s, 16)
    ones16 = jnp.ones((16,), jnp.float32)

    def fill_ones(i, _):
        vals[i, :] = ones16
        return 0

    lax.fori_loop(0, K, fill_ones, 0)
    plsc.subcore_barrier()

    ebase = wid * (CHUNKS * K)

    def chunk(i, _):
        off = ebase + i * K
        pltpu.sync_copy(dstp_hbm.at[pl.ds(off, K)], didx)
        pltpu.sync_copy(vals, acc.at[didx], add=True)
        return 0

    lax.fori_loop(0, CHUNKS, chunk, 0)
    plsc.subcore_barrier()
    base = s * ROWS_PER_TILE
    pltpu.sync_copy(acc.at[pl.ds(base, ROWS_PER_TILE)],
                    out_hbm.at[c, pl.ds(base, ROWS_PER_TILE)])


@functools.cache
def _deg_call():
    return pl.kernel(
        _deg_body,
        mesh=_sc_mesh(),
        out_type=jax.ShapeDtypeStruct((NC, NPAD, 16), jnp.float32),
        scratch_types=[
            pltpu.VMEM_SHARED((NPAD, 16), jnp.float32),
            pltpu.VMEM((K,), jnp.int32),
            pltpu.VMEM((K, 16), jnp.float32),
            pltpu.SemaphoreType.DMA,
        ],
    )


def _scatter_body(p_hbm, srcp_hbm, dstp_hbm, out_hbm, acc,
                  si0, si1, si2, si3, di0, di1, di2, di3,
                  rows0, rows1, is0, is1, is2, is3, gsem0, gsem1):
    c = lax.axis_index("c")
    s = lax.axis_index("s")
    nch = jnp.where(c == 0, CH0, CH1)
    ebase = jnp.where(c == 0, s * (CH0 * K), NS * CH0 * K + s * (CH1 * K))
    sbufs = (si0, si1, si2, si3)
    dbufs = (di0, di1, di2, di3)
    isems = (is0, is1, is2, is3)
    rbufs = (rows0, rows1)
    gsems = (gsem0, gsem1)

    def istart(t, q):
        off = ebase + t * K
        pltpu.make_async_copy(srcp_hbm.at[pl.ds(off, K)], sbufs[q], isems[q]).start()
        pltpu.make_async_copy(dstp_hbm.at[pl.ds(off, K)], dbufs[q], isems[q]).start()

    def iwait(q):
        pltpu.make_async_copy(srcp_hbm.at[pl.ds(0, K)], sbufs[q], isems[q]).wait()
        pltpu.make_async_copy(dstp_hbm.at[pl.ds(0, K)], dbufs[q], isems[q]).wait()

    def gstart(q, p):
        pltpu.make_async_copy(p_hbm.at[sbufs[q]], rbufs[p], gsems[p]).start()

    def gwait(p):
        pltpu.make_async_copy(p_hbm.at[sbufs[0]], rbufs[p], gsems[p]).wait()

    for q in range(4):
        istart(q, q)
    _zero_fill(rows0, K, F)
    _zero_acc_slice(acc, rows0, s, F)
    plsc.subcore_barrier()

    def body(m, _):
        t0 = 4 * m
        for q in range(4):
            p = q % 2
            t = t0 + q
            gwait(p)
            pltpu.sync_copy(rbufs[p], acc.at[dbufs[q]], add=True)

            @pl.when(t + 4 < nch)
            def _():
                istart(t + 4, q)

            q2 = (q + 2) % 4

            @pl.when(t + 2 < nch)
            def _():
                iwait(q2)
                gstart(q2, p)
        return 0

    iwait(0)
    gstart(0, 0)
    iwait(1)
    gstart(1, 1)
    lax.fori_loop(0, nch // 4, body, 0)
    plsc.subcore_barrier()
    base = s * ROWS_PER_TILE
    pltpu.sync_copy(acc.at[pl.ds(base, ROWS_PER_TILE)],
                    out_hbm.at[c, pl.ds(base, ROWS_PER_TILE)])


@functools.cache
def _scatter_call():
    idx = pltpu.VMEM((K,), jnp.int32)
    sem = pltpu.SemaphoreType.DMA
    return pl.kernel(
        _scatter_body,
        mesh=_sc_mesh(),
        out_type=jax.ShapeDtypeStruct((NC, NPAD, F), jnp.float32),
        scratch_types=[
            pltpu.VMEM_SHARED((NPAD, F), jnp.float32),
            idx, idx, idx, idx, idx, idx, idx, idx,
            pltpu.VMEM((K, F), jnp.float32),
            pltpu.VMEM((K, F), jnp.float32),
            sem, sem, sem, sem, sem, sem,
        ],
    )


def _prep_body(d0_ref, d1_ref, x_ref, w_ref, p_ref, dinv_ref):
    deg = d0_ref[...] + d1_ref[...] + 1.0
    dinv = lax.rsqrt(deg)
    dinv_ref[...] = dinv
    h = jnp.dot(x_ref[...], w_ref[...], preferred_element_type=jnp.float32)
    p_ref[...] = h * dinv


_prep_call = pl.pallas_call(
    _prep_body,
    out_shape=(
        jax.ShapeDtypeStruct((N, F), jnp.float32),
        jax.ShapeDtypeStruct((N, 1), jnp.float32),
    ),
)


def _bn(t, g, be):
    mu = jnp.mean(t, axis=0, keepdims=True)
    d = t - mu
    var = jnp.mean(d * d, axis=0, keepdims=True)
    return d * lax.rsqrt(var + 1e-5) * g + be


def _mid_body(acc_ref, p_ref, dinv_ref, b_ref, g_ref, be_ref, w_ref, out_ref):
    dinv = dinv_ref[...]
    t = (acc_ref[0, :N, :] + acc_ref[1, :N, :] + p_ref[...]) * dinv + b_ref[...]
    h = jnp.maximum(_bn(t, g_ref[...], be_ref[...]), 0.0)
    out_ref[...] = jnp.dot(h, w_ref[...], preferred_element_type=jnp.float32) * dinv


_mid_call = pl.pallas_call(
    _mid_body,
    out_shape=jax.ShapeDtypeStruct((N, F), jnp.float32),
)


def _final_body(acc_ref, p_ref, dinv_ref, b_ref, g_ref, be_ref, batch_ref, out_ref):
    t = (acc_ref[0, :N, :] + acc_ref[1, :N, :] + p_ref[...]) * dinv_ref[...] + b_ref[...]
    h = _bn(t, g_ref[...], be_ref[...])
    gid = lax.broadcasted_iota(jnp.int32, (N, G), 1)
    onehot = (batch_ref[...] == gid).astype(jnp.float32)
    dims = (((0,), (0,)), ((), ()))
    sums = lax.dot_general(onehot, h, dims, preferred_element_type=jnp.float32)
    cnt = lax.dot_general(onehot, jnp.ones((N, 1), jnp.float32), dims,
                          preferred_element_type=jnp.float32)
    out_ref[...] = sums / jnp.maximum(cnt, 1.0)


_final_call = pl.pallas_call(
    _final_body,
    out_shape=jax.ShapeDtypeStruct((G, F), jnp.float32),
)


def kernel(x, edge_index, batch, W1, b1, g1, be1, W2, b2, g2, be2, W3, b3, g3, be3):
    pad = EPAD - E
    srcp = jnp.concatenate([edge_index[0], jnp.zeros((pad,), jnp.int32)])
    dstp = jnp.concatenate([edge_index[1], jnp.full((pad,), N, jnp.int32)])

    degparts = _deg_call()(dstp)
    d0 = degparts[0, :N, 0:1]
    d1 = degparts[1, :N, 0:1]

    b1r, g1r, be1r = b1[None, :], g1[None, :], be1[None, :]
    b2r, g2r, be2r = b2[None, :], g2[None, :], be2[None, :]
    b3r, g3r, be3r = b3[None, :], g3[None, :], be3[None, :]

    scatter = _scatter_call()
    p1, dinv = _prep_call(d0, d1, x, W1)
    s1 = scatter(p1, srcp, dstp)
    p2 = _mid_call(s1, p1, dinv, b1r, g1r, be1r, W2)
    s2 = scatter(p2, srcp, dstp)
    p3 = _mid_call(s2, p2, dinv, b2r, g2r, be2r, W3)
    s3 = scatter(p3, srcp, dstp)
    return _final_call(s3, p3, dinv, b3r, g3r, be3r, batch[:, None])


# speedup vs baseline: 1.0650x; 1.0650x over previous
import functools

import jax
import jax.numpy as jnp
import numpy as np
from jax import lax
from jax.experimental import pallas as pl
from jax.experimental.pallas import tpu as pltpu
from jax.experimental.pallas import tpu_sc as plsc

N = 10000
E = 320000
F = 128
G = 64
NC = 2
NS = 16
K = 128
CHUNKS = (-(-E // (NC * NS * K)) + 3) // 4 * 4
CH0 = 156
CH1 = 2 * CHUNKS - CH0
EPAD = NC * NS * K * CHUNKS
NPAD = N + 112
ROWS_PER_TILE = NPAD // NS

@functools.cache
def _sc_mesh():
    return plsc.VectorSubcoreMesh(core_axis_name="c", subcore_axis_name="s")


def _zero_fill(ref, nrows, ncols):
    zeros16 = jnp.zeros((16,), jnp.float32)

    def body(i, _):
        ref[i // (ncols // 16), pl.ds((i % (ncols // 16)) * 16, 16)] = zeros16
        return 0

    lax.fori_loop(0, nrows * (ncols // 16), body, 0)


def _zero_fill3(ref, n0, n1):
    zeros16 = jnp.zeros((16,), jnp.float32)

    def body(i, _):
        ref[i // n1, i % n1, :] = zeros16
        return 0

    lax.fori_loop(0, n0 * n1, body, 0)


def _zero_acc_slice(acc, rows_buf, tile, width):
    base = tile * ROWS_PER_TILE
    done = 0
    while done < ROWS_PER_TILE:
        nr = min(K, ROWS_PER_TILE - done)
        pltpu.sync_copy(rows_buf.at[pl.ds(0, nr)], acc.at[pl.ds(base + done, nr)])
        done += nr


def _deg_body(dstp_hbm, out_hbm, acc, didx, vals, sem):
    c = lax.axis_index("c")
    s = lax.axis_index("s")
    wid = c * NS + s
    _zero_fill(vals, K, 16)
    _zero_acc_slice(acc, vals, s, 16)
    ones16 = jnp.ones((16,), jnp.float32)

    def fill_ones(i, _):
        vals[i, :] = ones16
        return 0

    lax.fori_loop(0, K, fill_ones, 0)
    plsc.subcore_barrier()

    ebase = wid * (CHUNKS * K)

    def chunk(i, _):
        off = ebase + i * K
        pltpu.sync_copy(dstp_hbm.at[pl.ds(off, K)], didx)
        pltpu.sync_copy(vals, acc.at[didx], add=True)
        return 0

    lax.fori_loop(0, CHUNKS, chunk, 0)
    plsc.subcore_barrier()
    base = s * ROWS_PER_TILE
    pltpu.sync_copy(acc.at[pl.ds(base, ROWS_PER_TILE)],
                    out_hbm.at[c, pl.ds(base, ROWS_PER_TILE)])


@functools.cache
def _deg_call():
    return pl.kernel(
        _deg_body,
        mesh=_sc_mesh(),
        out_type=jax.ShapeDtypeStruct((NC, NPAD, 16), jnp.float32),
        scratch_types=[
            pltpu.VMEM_SHARED((NPAD, 16), jnp.float32),
            pltpu.VMEM((K,), jnp.int32),
            pltpu.VMEM((K, 16), jnp.float32),
            pltpu.SemaphoreType.DMA,
        ],
    )


def _scatter_body(p_hbm, srcp_hbm, dstp_hbm, out_hbm, acc,
                  si0, si1, si2, si3, di0, di1, di2, di3,
                  rows0, rows1, is0, is1, is2, is3, gsem0, gsem1):
    c = lax.axis_index("c")
    s = lax.axis_index("s")
    nch = jnp.where(c == 0, CH0, CH1)
    ebase = jnp.where(c == 0, s * (CH0 * K), NS * CH0 * K + s * (CH1 * K))
    sbufs = (si0, si1, si2, si3)
    dbufs = (di0, di1, di2, di3)
    isems = (is0, is1, is2, is3)
    rbufs = (rows0, rows1)
    gsems = (gsem0, gsem1)

    def istart(t, q):
        off = ebase + t * K
        pltpu.make_async_copy(srcp_hbm.at[pl.ds(off, K)], sbufs[q], isems[q]).start()
        pltpu.make_async_copy(dstp_hbm.at[pl.ds(off, K)], dbufs[q], isems[q]).start()

    def iwait(q):
        pltpu.make_async_copy(srcp_hbm.at[pl.ds(0, K)], sbufs[q], isems[q]).wait()
        pltpu.make_async_copy(dstp_hbm.at[pl.ds(0, K)], dbufs[q], isems[q]).wait()

    def gstart(q, p):
        pltpu.make_async_copy(p_hbm.at[sbufs[q]], rbufs[p], gsems[p]).start()

    def gwait(p):
        pltpu.make_async_copy(p_hbm.at[sbufs[0]], rbufs[p], gsems[p]).wait()

    for q in range(4):
        istart(q, q)
    _zero_fill(rows0, K, F)
    _zero_acc_slice(acc, rows0, s, F)
    plsc.subcore_barrier()

    def body(m, _):
        t0 = 4 * m
        for q in range(4):
            p = q % 2
            t = t0 + q
            gwait(p)
            pltpu.sync_copy(rbufs[p], acc.at[dbufs[q]], add=True)

            @pl.when(t + 4 < nch)
            def _():
                istart(t + 4, q)

            q2 = (q + 2) % 4

            @pl.when(t + 2 < nch)
            def _():
                iwait(q2)
                gstart(q2, p)
        return 0

    iwait(0)
    gstart(0, 0)
    iwait(1)
    gstart(1, 1)
    lax.fori_loop(0, nch // 4, body, 0)
    plsc.subcore_barrier()
    base = s * ROWS_PER_TILE
    pltpu.sync_copy(acc.at[pl.ds(base, ROWS_PER_TILE)],
                    out_hbm.at[c, pl.ds(base, ROWS_PER_TILE)])


@functools.cache
def _scatter_call():
    idx = pltpu.VMEM((K,), jnp.int32)
    sem = pltpu.SemaphoreType.DMA
    return pl.kernel(
        _scatter_body,
        mesh=_sc_mesh(),
        out_type=jax.ShapeDtypeStruct((NC, NPAD, F), jnp.float32),
        scratch_types=[
            pltpu.VMEM_SHARED((NPAD, F), jnp.float32),
            idx, idx, idx, idx, idx, idx, idx, idx,
            pltpu.VMEM((K, F), jnp.float32),
            pltpu.VMEM((K, F), jnp.float32),
            sem, sem, sem, sem, sem, sem,
        ],
    )


def _prep_body(d0_ref, d1_ref, x_ref, w_ref, p_ref, dinv_ref):
    deg = d0_ref[...] + d1_ref[...] + 1.0
    dinv = lax.rsqrt(deg)
    dinv_ref[...] = dinv
    h = jnp.dot(x_ref[...], w_ref[...], preferred_element_type=jnp.float32)
    p_ref[...] = h * dinv


_prep_call = pl.pallas_call(
    _prep_body,
    out_shape=(
        jax.ShapeDtypeStruct((N, F), jnp.float32),
        jax.ShapeDtypeStruct((N, 1), jnp.float32),
    ),
)


def _bn(t, g, be):
    mu = jnp.mean(t, axis=0, keepdims=True)
    d = t - mu
    var = jnp.mean(d * d, axis=0, keepdims=True)
    return d * lax.rsqrt(var + 1e-5) * g + be


def _mid_body(acc_ref, p_ref, dinv_ref, b_ref, g_ref, be_ref, w_ref, out_ref):
    dinv = dinv_ref[...]
    t = (acc_ref[0, :N, :] + acc_ref[1, :N, :] + p_ref[...]) * dinv + b_ref[...]
    h = jnp.maximum(_bn(t, g_ref[...], be_ref[...]), 0.0)
    out_ref[...] = jnp.dot(h, w_ref[...], preferred_element_type=jnp.float32) * dinv


_mid_call = pl.pallas_call(
    _mid_body,
    out_shape=jax.ShapeDtypeStruct((N, F), jnp.float32),
)


def _final_body(acc_ref, p_ref, dinv_ref, b_ref, g_ref, be_ref, batch_ref, out_ref):
    t = (acc_ref[0, :N, :] + acc_ref[1, :N, :] + p_ref[...]) * dinv_ref[...] + b_ref[...]
    h = _bn(t, g_ref[...], be_ref[...])
    gid = lax.broadcasted_iota(jnp.int32, (N, G), 1)
    onehot = (batch_ref[...] == gid).astype(jnp.float32)
    dims = (((0,), (0,)), ((), ()))
    sums = lax.dot_general(onehot, h, dims, preferred_element_type=jnp.float32)
    cnt = lax.dot_general(onehot, jnp.ones((N, 1), jnp.float32), dims,
                          preferred_element_type=jnp.float32)
    out_ref[...] = sums / jnp.maximum(cnt, 1.0)


_final_call = pl.pallas_call(
    _final_body,
    out_shape=jax.ShapeDtypeStruct((G, F), jnp.float32),
)


def kernel(x, edge_index, batch, W1, b1, g1, be1, W2, b2, g2, be2, W3, b3, g3, be3):
    pad = EPAD - E
    srcp = jnp.concatenate([edge_index[0], jnp.zeros((pad,), jnp.int32)])
    dstp = jnp.concatenate([edge_index[1], jnp.full((pad,), N, jnp.int32)])

    degparts = _deg_call()(dstp)
    d0 = degparts[0, :N, 0:1]
    d1 = degparts[1, :N, 0:1]

    b1r, g1r, be1r = b1[None, :], g1[None, :], be1[None, :]
    b2r, g2r, be2r = b2[None, :], g2[None, :], be2[None, :]
    b3r, g3r, be3r = b3[None, :], g3[None, :], be3[None, :]

    scatter = _scatter_call()
    p1, dinv = _prep_call(d0, d1, x, W1)
    s1 = scatter(p1, srcp, dstp)
    p2 = _mid_call(s1, p1, dinv, b1r, g1r, be1r, W2)
    s2 = scatter(p2, srcp, dstp)
    p3 = _mid_call(s2, p2, dinv, b2r, g2r, be2r, W3)
    s3 = scatter(p3, srcp, dstp)
    return _final_call(s3, p3, dinv, b3r, g3r, be3r, batch[:, None])


# speedup vs baseline: 1.0661x; 1.0011x over previous
import functools

import jax
import jax.numpy as jnp
import numpy as np
from jax import lax
from jax.experimental import pallas as pl
from jax.experimental.pallas import tpu as pltpu
from jax.experimental.pallas import tpu_sc as plsc

N = 10000
E = 320000
F = 128
G = 64
NC = 2
NS = 16
K = 128
CHUNKS = (-(-E // (NC * NS * K)) + 3) // 4 * 4
CH0 = 144
CH1 = 2 * CHUNKS - CH0
EPAD = NC * NS * K * CHUNKS
NPAD = N + 112
ROWS_PER_TILE = NPAD // NS

@functools.cache
def _sc_mesh():
    return plsc.VectorSubcoreMesh(core_axis_name="c", subcore_axis_name="s")


def _zero_fill(ref, nrows, ncols):
    zeros16 = jnp.zeros((16,), jnp.float32)

    def body(i, _):
        ref[i // (ncols // 16), pl.ds((i % (ncols // 16)) * 16, 16)] = zeros16
        return 0

    lax.fori_loop(0, nrows * (ncols // 16), body, 0)


def _zero_fill3(ref, n0, n1):
    zeros16 = jnp.zeros((16,), jnp.float32)

    def body(i, _):
        ref[i // n1, i % n1, :] = zeros16
        return 0

    lax.fori_loop(0, n0 * n1, body, 0)


def _zero_acc_slice(acc, rows_buf, tile, width):
    base = tile * ROWS_PER_TILE
    done = 0
    while done < ROWS_PER_TILE:
        nr = min(K, ROWS_PER_TILE - done)
        pltpu.sync_copy(rows_buf.at[pl.ds(0, nr)], acc.at[pl.ds(base + done, nr)])
        done += nr


def _deg_body(dstp_hbm, out_hbm, acc, didx, vals, sem):
    c = lax.axis_index("c")
    s = lax.axis_index("s")
    wid = c * NS + s
    _zero_fill(vals, K, 16)
    _zero_acc_slice(acc, vals, s, 16)
    ones16 = jnp.ones((16,), jnp.float32)

    def fill_ones(i, _):
        vals[i, :] = ones16
        return 0

    lax.fori_loop(0, K, fill_ones, 0)
    plsc.subcore_barrier()

    ebase = wid * (CHUNKS * K)

    def chunk(i, _):
        off = ebase + i * K
        pltpu.sync_copy(dstp_hbm.at[pl.ds(off, K)], didx)
        pltpu.sync_copy(vals, acc.at[didx], add=True)
        return 0

    lax.fori_loop(0, CHUNKS, chunk, 0)
    plsc.subcore_barrier()
    base = s * ROWS_PER_TILE
    pltpu.sync_copy(acc.at[pl.ds(base, ROWS_PER_TILE)],
                    out_hbm.at[c, pl.ds(base, ROWS_PER_TILE)])


@functools.cache
def _deg_call():
    return pl.kernel(
        _deg_body,
        mesh=_sc_mesh(),
        out_type=jax.ShapeDtypeStruct((NC, NPAD, 16), jnp.float32),
        scratch_types=[
            pltpu.VMEM_SHARED((NPAD, 16), jnp.float32),
            pltpu.VMEM((K,), jnp.int32),
            pltpu.VMEM((K, 16), jnp.float32),
            pltpu.SemaphoreType.DMA,
        ],
    )


def _scatter_body(p_hbm, srcp_hbm, dstp_hbm, out_hbm, acc,
                  si0, si1, si2, si3, di0, di1, di2, di3,
                  rows0, rows1, is0, is1, is2, is3, gsem0, gsem1):
    c = lax.axis_index("c")
    s = lax.axis_index("s")
    nch = jnp.where(c == 0, CH0, CH1)
    ebase = jnp.where(c == 0, s * (CH0 * K), NS * CH0 * K + s * (CH1 * K))
    sbufs = (si0, si1, si2, si3)
    dbufs = (di0, di1, di2, di3)
    isems = (is0, is1, is2, is3)
    rbufs = (rows0, rows1)
    gsems = (gsem0, gsem1)

    def istart(t, q):
        off = ebase + t * K
        pltpu.make_async_copy(srcp_hbm.at[pl.ds(off, K)], sbufs[q], isems[q]).start()
        pltpu.make_async_copy(dstp_hbm.at[pl.ds(off, K)], dbufs[q], isems[q]).start()

    def iwait(q):
        pltpu.make_async_copy(srcp_hbm.at[pl.ds(0, K)], sbufs[q], isems[q]).wait()
        pltpu.make_async_copy(dstp_hbm.at[pl.ds(0, K)], dbufs[q], isems[q]).wait()

    def gstart(q, p):
        pltpu.make_async_copy(p_hbm.at[sbufs[q]], rbufs[p], gsems[p]).start()

    def gwait(p):
        pltpu.make_async_copy(p_hbm.at[sbufs[0]], rbufs[p], gsems[p]).wait()

    for q in range(4):
        istart(q, q)
    _zero_fill(rows0, K, F)
    _zero_acc_slice(acc, rows0, s, F)
    plsc.subcore_barrier()

    def body(m, _):
        t0 = 4 * m
        for q in range(4):
            p = q % 2
            t = t0 + q
            gwait(p)
            pltpu.sync_copy(rbufs[p], acc.at[dbufs[q]], add=True)

            @pl.when(t + 4 < nch)
            def _():
                istart(t + 4, q)

            q2 = (q + 2) % 4

            @pl.when(t + 2 < nch)
            def _():
                iwait(q2)
                gstart(q2, p)
        return 0

    iwait(0)
    gstart(0, 0)
    iwait(1)
    gstart(1, 1)
    lax.fori_loop(0, nch // 4, body, 0)
    plsc.subcore_barrier()
    base = s * ROWS_PER_TILE
    pltpu.sync_copy(acc.at[pl.ds(base, ROWS_PER_TILE)],
                    out_hbm.at[c, pl.ds(base, ROWS_PER_TILE)])


@functools.cache
def _scatter_call():
    idx = pltpu.VMEM((K,), jnp.int32)
    sem = pltpu.SemaphoreType.DMA
    return pl.kernel(
        _scatter_body,
        mesh=_sc_mesh(),
        out_type=jax.ShapeDtypeStruct((NC, NPAD, F), jnp.float32),
        scratch_types=[
            pltpu.VMEM_SHARED((NPAD, F), jnp.float32),
            idx, idx, idx, idx, idx, idx, idx, idx,
            pltpu.VMEM((K, F), jnp.float32),
            pltpu.VMEM((K, F), jnp.float32),
            sem, sem, sem, sem, sem, sem,
        ],
    )


def _prep_body(d0_ref, d1_ref, x_ref, w_ref, p_ref, dinv_ref):
    deg = d0_ref[...] + d1_ref[...] + 1.0
    dinv = lax.rsqrt(deg)
    dinv_ref[...] = dinv
    h = jnp.dot(x_ref[...], w_ref[...], preferred_element_type=jnp.float32)
    p_ref[...] = h * dinv


_prep_call = pl.pallas_call(
    _prep_body,
    out_shape=(
        jax.ShapeDtypeStruct((N, F), jnp.float32),
        jax.ShapeDtypeStruct((N, 1), jnp.float32),
    ),
)


def _bn(t, g, be):
    mu = jnp.mean(t, axis=0, keepdims=True)
    d = t - mu
    var = jnp.mean(d * d, axis=0, keepdims=True)
    return d * lax.rsqrt(var + 1e-5) * g + be


def _mid_body(acc_ref, p_ref, dinv_ref, b_ref, g_ref, be_ref, w_ref, out_ref):
    dinv = dinv_ref[...]
    t = (acc_ref[0, :N, :] + acc_ref[1, :N, :] + p_ref[...]) * dinv + b_ref[...]
    h = jnp.maximum(_bn(t, g_ref[...], be_ref[...]), 0.0)
    out_ref[...] = jnp.dot(h, w_ref[...], preferred_element_type=jnp.float32) * dinv


_mid_call = pl.pallas_call(
    _mid_body,
    out_shape=jax.ShapeDtypeStruct((N, F), jnp.float32),
)


def _final_body(acc_ref, p_ref, dinv_ref, b_ref, g_ref, be_ref, batch_ref, out_ref):
    t = (acc_ref[0, :N, :] + acc_ref[1, :N, :] + p_ref[...]) * dinv_ref[...] + b_ref[...]
    h = _bn(t, g_ref[...], be_ref[...])
    gid = lax.broadcasted_iota(jnp.int32, (N, G), 1)
    onehot = (batch_ref[...] == gid).astype(jnp.float32)
    dims = (((0,), (0,)), ((), ()))
    sums = lax.dot_general(onehot, h, dims, preferred_element_type=jnp.float32)
    cnt = lax.dot_general(onehot, jnp.ones((N, 1), jnp.float32), dims,
                          preferred_element_type=jnp.float32)
    out_ref[...] = sums / jnp.maximum(cnt, 1.0)


_final_call = pl.pallas_call(
    _final_body,
    out_shape=jax.ShapeDtypeStruct((G, F), jnp.float32),
)


def kernel(x, edge_index, batch, W1, b1, g1, be1, W2, b2, g2, be2, W3, b3, g3, be3):
    pad = EPAD - E
    srcp = jnp.concatenate([edge_index[0], jnp.zeros((pad,), jnp.int32)])
    dstp = jnp.concatenate([edge_index[1], jnp.full((pad,), N, jnp.int32)])

    degparts = _deg_call()(dstp)
    d0 = degparts[0, :N, 0:1]
    d1 = degparts[1, :N, 0:1]

    b1r, g1r, be1r = b1[None, :], g1[None, :], be1[None, :]
    b2r, g2r, be2r = b2[None, :], g2[None, :], be2[None, :]
    b3r, g3r, be3r = b3[None, :], g3[None, :], be3[None, :]

    scatter = _scatter_call()
    p1, dinv = _prep_call(d0, d1, x, W1)
    s1 = scatter(p1, srcp, dstp)
    p2 = _mid_call(s1, p1, dinv, b1r, g1r, be1r, W2)
    s2 = scatter(p2, srcp, dstp)
    p3 = _mid_call(s2, p2, dinv, b2r, g2r, be2r, W3)
    s3 = scatter(p3, srcp, dstp)
    return _final_call(s3, p3, dinv, b3r, g3r, be3r, batch[:, None])


# speedup vs baseline: 1.0662x; 1.0001x over previous
import functools

import jax
import jax.numpy as jnp
from jax import lax
from jax.experimental import pallas as pl
from jax.experimental.pallas import tpu as pltpu
from jax.experimental.pallas import tpu_sc as plsc

N = 10000
E = 320000
F = 128
G = 64
NC = 2
NS = 16
K = 128
CHUNKS = (-(-E // (NC * NS * K)) + 3) // 4 * 4
CH0 = 144
CH1 = 2 * CHUNKS - CH0
EPAD = NC * NS * K * CHUNKS
NPAD = N + 112
ROWS_PER_TILE = NPAD // NS

@functools.cache
def _sc_mesh():
    return plsc.VectorSubcoreMesh(core_axis_name="c", subcore_axis_name="s")


def _zero_fill(ref, nrows, ncols):
    zeros16 = jnp.zeros((16,), jnp.float32)

    def body(i, _):
        ref[i // (ncols // 16), pl.ds((i % (ncols // 16)) * 16, 16)] = zeros16
        return 0

    lax.fori_loop(0, nrows * (ncols // 16), body, 0)


def _zero_acc_slice(acc, rows_buf, tile, width):
    base = tile * ROWS_PER_TILE
    done = 0
    while done < ROWS_PER_TILE:
        nr = min(K, ROWS_PER_TILE - done)
        pltpu.sync_copy(rows_buf.at[pl.ds(0, nr)], acc.at[pl.ds(base + done, nr)])
        done += nr


def _deg_body(dstp_hbm, out_hbm, acc, didx, vals, sem):
    c = lax.axis_index("c")
    s = lax.axis_index("s")
    wid = c * NS + s
    _zero_fill(vals, K, 16)
    _zero_acc_slice(acc, vals, s, 16)
    ones16 = jnp.ones((16,), jnp.float32)

    def fill_ones(i, _):
        vals[i, :] = ones16
        return 0

    lax.fori_loop(0, K, fill_ones, 0)
    plsc.subcore_barrier()

    ebase = wid * (CHUNKS * K)

    def chunk(i, _):
        off = ebase + i * K
        pltpu.sync_copy(dstp_hbm.at[pl.ds(off, K)], didx)
        pltpu.sync_copy(vals, acc.at[didx], add=True)
        return 0

    lax.fori_loop(0, CHUNKS, chunk, 0)
    plsc.subcore_barrier()
    base = s * ROWS_PER_TILE
    pltpu.sync_copy(acc.at[pl.ds(base, ROWS_PER_TILE)],
                    out_hbm.at[c, pl.ds(base, ROWS_PER_TILE)])


@functools.cache
def _deg_call():
    return pl.kernel(
        _deg_body,
        mesh=_sc_mesh(),
        out_type=jax.ShapeDtypeStruct((NC, NPAD, 16), jnp.float32),
        scratch_types=[
            pltpu.VMEM_SHARED((NPAD, 16), jnp.float32),
            pltpu.VMEM((K,), jnp.int32),
            pltpu.VMEM((K, 16), jnp.float32),
            pltpu.SemaphoreType.DMA,
        ],
    )


def _scatter_body(p_hbm, srcp_hbm, dstp_hbm, out_hbm, acc,
                  si0, si1, si2, si3, di0, di1, di2, di3,
                  rows0, rows1, is0, is1, is2, is3, gsem0, gsem1):
    c = lax.axis_index("c")
    s = lax.axis_index("s")
    nch = jnp.where(c == 0, CH0, CH1)
    ebase = jnp.where(c == 0, s * (CH0 * K), NS * CH0 * K + s * (CH1 * K))
    sbufs = (si0, si1, si2, si3)
    dbufs = (di0, di1, di2, di3)
    isems = (is0, is1, is2, is3)
    rbufs = (rows0, rows1)
    gsems = (gsem0, gsem1)

    def istart(t, q):
        off = ebase + t * K
        pltpu.make_async_copy(srcp_hbm.at[pl.ds(off, K)], sbufs[q], isems[q]).start()
        pltpu.make_async_copy(dstp_hbm.at[pl.ds(off, K)], dbufs[q], isems[q]).start()

    def iwait(q):
        pltpu.make_async_copy(srcp_hbm.at[pl.ds(0, K)], sbufs[q], isems[q]).wait()
        pltpu.make_async_copy(dstp_hbm.at[pl.ds(0, K)], dbufs[q], isems[q]).wait()

    def gstart(q, p):
        pltpu.make_async_copy(p_hbm.at[sbufs[q]], rbufs[p], gsems[p]).start()

    def gwait(p):
        pltpu.make_async_copy(p_hbm.at[sbufs[0]], rbufs[p], gsems[p]).wait()

    for q in range(4):
        istart(q, q)
    _zero_fill(rows0, K, F)
    _zero_acc_slice(acc, rows0, s, F)
    plsc.subcore_barrier()

    def body(m, _):
        t0 = 4 * m
        for q in range(4):
            p = q % 2
            t = t0 + q
            gwait(p)
            pltpu.sync_copy(rbufs[p], acc.at[dbufs[q]], add=True)

            @pl.when(t + 4 < nch)
            def _():
                istart(t + 4, q)

            q2 = (q + 2) % 4

            @pl.when(t + 2 < nch)
            def _():
                iwait(q2)
                gstart(q2, p)
        return 0

    iwait(0)
    gstart(0, 0)
    iwait(1)
    gstart(1, 1)
    lax.fori_loop(0, nch // 4, body, 0)
    plsc.subcore_barrier()
    base = s * ROWS_PER_TILE
    pltpu.sync_copy(acc.at[pl.ds(base, ROWS_PER_TILE)],
                    out_hbm.at[c, pl.ds(base, ROWS_PER_TILE)])


@functools.cache
def _scatter_call():
    idx = pltpu.VMEM((K,), jnp.int32)
    sem = pltpu.SemaphoreType.DMA
    return pl.kernel(
        _scatter_body,
        mesh=_sc_mesh(),
        out_type=jax.ShapeDtypeStruct((NC, NPAD, F), jnp.float32),
        scratch_types=[
            pltpu.VMEM_SHARED((NPAD, F), jnp.float32),
            idx, idx, idx, idx, idx, idx, idx, idx,
            pltpu.VMEM((K, F), jnp.float32),
            pltpu.VMEM((K, F), jnp.float32),
            sem, sem, sem, sem, sem, sem,
        ],
    )


def _prep_body(d0_ref, d1_ref, x_ref, w_ref, p_ref, dinv_ref):
    deg = d0_ref[...] + d1_ref[...] + 1.0
    dinv = lax.rsqrt(deg)
    dinv_ref[...] = dinv
    h = jnp.dot(x_ref[...], w_ref[...], preferred_element_type=jnp.float32)
    p_ref[...] = h * dinv


_prep_call = pl.pallas_call(
    _prep_body,
    out_shape=(
        jax.ShapeDtypeStruct((N, F), jnp.float32),
        jax.ShapeDtypeStruct((N, 1), jnp.float32),
    ),
)


def _bn(t, g, be):
    mu = jnp.mean(t, axis=0, keepdims=True)
    d = t - mu
    var = jnp.mean(d * d, axis=0, keepdims=True)
    return d * lax.rsqrt(var + 1e-5) * g + be


def _mid_body(acc_ref, p_ref, dinv_ref, b_ref, g_ref, be_ref, w_ref, out_ref):
    dinv = dinv_ref[...]
    t = (acc_ref[0, :N, :] + acc_ref[1, :N, :] + p_ref[...]) * dinv + b_ref[...]
    h = jnp.maximum(_bn(t, g_ref[...], be_ref[...]), 0.0)
    out_ref[...] = jnp.dot(h, w_ref[...], preferred_element_type=jnp.float32) * dinv


_mid_call = pl.pallas_call(
    _mid_body,
    out_shape=jax.ShapeDtypeStruct((N, F), jnp.float32),
)


def _final_body(acc_ref, p_ref, dinv_ref, b_ref, g_ref, be_ref, batch_ref, out_ref):
    t = (acc_ref[0, :N, :] + acc_ref[1, :N, :] + p_ref[...]) * dinv_ref[...] + b_ref[...]
    h = _bn(t, g_ref[...], be_ref[...])
    gid = lax.broadcasted_iota(jnp.int32, (N, G), 1)
    onehot = (batch_ref[...] == gid).astype(jnp.float32)
    dims = (((0,), (0,)), ((), ()))
    sums = lax.dot_general(onehot, h, dims, preferred_element_type=jnp.float32)
    cnt = lax.dot_general(onehot, jnp.ones((N, 1), jnp.float32), dims,
                          preferred_element_type=jnp.float32)
    out_ref[...] = sums / jnp.maximum(cnt, 1.0)


_final_call = pl.pallas_call(
    _final_body,
    out_shape=jax.ShapeDtypeStruct((G, F), jnp.float32),
)


def kernel(x, edge_index, batch, W1, b1, g1, be1, W2, b2, g2, be2, W3, b3, g3, be3):
    pad = EPAD - E
    srcp = jnp.concatenate([edge_index[0], jnp.zeros((pad,), jnp.int32)])
    dstp = jnp.concatenate([edge_index[1], jnp.full((pad,), N, jnp.int32)])

    degparts = _deg_call()(dstp)
    d0 = degparts[0, :N, 0:1]
    d1 = degparts[1, :N, 0:1]

    b1r, g1r, be1r = b1[None, :], g1[None, :], be1[None, :]
    b2r, g2r, be2r = b2[None, :], g2[None, :], be2[None, :]
    b3r, g3r, be3r = b3[None, :], g3[None, :], be3[None, :]

    scatter = _scatter_call()
    p1, dinv = _prep_call(d0, d1, x, W1)
    s1 = scatter(p1, srcp, dstp)
    p2 = _mid_call(s1, p1, dinv, b1r, g1r, be1r, W2)
    s2 = scatter(p2, srcp, dstp)
    p3 = _mid_call(s2, p2, dinv, b2r, g2r, be2r, W3)
    s3 = scatter(p3, srcp, dstp)
    return _final_call(s3, p3, dinv, b3r, g3r, be3r, batch[:, None])
